# baseline (speedup 1.0000x reference)
import functools

import jax
import jax.numpy as jnp
from jax import lax
from jax.experimental import pallas as pl
from jax.experimental.pallas import tpu as pltpu
from jax.experimental.pallas import tpu_sc as plsc

NC = 2
NS = 16
L = 16
NW = NC * NS

B = 16384
D = 128
BPW = B // NW
CHUNK = 64
NCHUNK = BPW // CHUNK


def _body(s_h, r_h, o_h, ere_h, eim_h, rre_h, rim_h, out_h,
          idx_s, idx_r, idx_o,
          sre_b, sim_b, ore_b, oim_b, rre_b, rim_b,
          out_v, sem):
    wid = lax.axis_index("s") * NC + lax.axis_index("c")
    base = wid * BPW

    pltpu.sync_copy(s_h.at[pl.ds(base, BPW)], idx_s)
    pltpu.sync_copy(r_h.at[pl.ds(base, BPW)], idx_r)
    pltpu.sync_copy(o_h.at[pl.ds(base, BPW)], idx_o)

    lane = lax.iota(jnp.int32, L)

    for c in range(NCHUNK):
        isl = pl.ds(c * CHUNK, CHUNK)
        d0 = pltpu.async_copy(ere_h.at[idx_s.at[isl]], sre_b, sem)
        d1 = pltpu.async_copy(eim_h.at[idx_s.at[isl]], sim_b, sem)
        d2 = pltpu.async_copy(ere_h.at[idx_o.at[isl]], ore_b, sem)
        d3 = pltpu.async_copy(eim_h.at[idx_o.at[isl]], oim_b, sem)
        d4 = pltpu.async_copy(rre_h.at[idx_r.at[isl]], rre_b, sem)
        d5 = pltpu.async_copy(rim_h.at[idx_r.at[isl]], rim_b, sem)
        d0.wait(); d1.wait(); d2.wait(); d3.wait(); d4.wait(); d5.wait()

        for g in range(CHUNK // L):
            rows = lane + (g * L)

            def dim_body(d, acc):
                col = jnp.zeros((L,), jnp.int32) + d
                sre = plsc.load_gather(sre_b, [rows, col])
                sim = plsc.load_gather(sim_b, [rows, col])
                orr = plsc.load_gather(ore_b, [rows, col])
                oim = plsc.load_gather(oim_b, [rows, col])
                rr = plsc.load_gather(rre_b, [rows, col])
                ri = plsc.load_gather(rim_b, [rows, col])
                t1 = rr * orr + ri * oim
                t2 = rr * oim - ri * orr
                return acc + sre * t1 + sim * t2

            acc = lax.fori_loop(0, D, dim_body, jnp.zeros((L,), jnp.float32))
            out_v[pl.ds(c * CHUNK + g * L, L)] = acc

    pltpu.sync_copy(out_v, out_h.at[pl.ds(base, BPW)])


@jax.jit
def _score(s, r, o, E_re, E_im, R_re, R_im):
    mesh = plsc.VectorSubcoreMesh(core_axis_name="c", subcore_axis_name="s")
    f = functools.partial(
        pl.kernel,
        out_type=jax.ShapeDtypeStruct((B,), jnp.float32),
        mesh=mesh,
        compiler_params=pltpu.CompilerParams(needs_layout_passes=False),
        scratch_types=[
            pltpu.VMEM((BPW,), jnp.int32),
            pltpu.VMEM((BPW,), jnp.int32),
            pltpu.VMEM((BPW,), jnp.int32),
            pltpu.VMEM((CHUNK, D), jnp.float32),
            pltpu.VMEM((CHUNK, D), jnp.float32),
            pltpu.VMEM((CHUNK, D), jnp.float32),
            pltpu.VMEM((CHUNK, D), jnp.float32),
            pltpu.VMEM((CHUNK, D), jnp.float32),
            pltpu.VMEM((CHUNK, D), jnp.float32),
            pltpu.VMEM((BPW,), jnp.float32),
            pltpu.SemaphoreType.DMA,
        ],
    )(_body)
    return f(s, r, o, E_re, E_im, R_re, R_im)


def kernel(s, r, o, t, E_re, E_im, R_re, R_im):
    del t
    return _score(s, r, o, E_re, E_im, R_re, R_im)


# speedup vs baseline: 1.2360x; 1.2360x over previous
import functools

import jax
import jax.numpy as jnp
from jax import lax
from jax.experimental import pallas as pl
from jax.experimental.pallas import tpu as pltpu
from jax.experimental.pallas import tpu_sc as plsc

NC = 2
NS = 16
L = 16
NW = NC * NS

B = 16384
D = 128
BPW = B // NW
CHUNK = 64
NCHUNK = BPW // CHUNK


UNROLL = 4


def _body(s_h, r_h, o_h, ere_h, eim_h, rre_h, rim_h, out_h,
          idx_s, idx_r, idx_o,
          sre_b, sim_b, ore_b, oim_b, rre_b, rim_b,
          out_v, sem):
    wid = lax.axis_index("s") * NC + lax.axis_index("c")
    base = wid * BPW

    pltpu.sync_copy(s_h.at[pl.ds(base, BPW)], idx_s)
    pltpu.sync_copy(r_h.at[pl.ds(base, BPW)], idx_r)
    pltpu.sync_copy(o_h.at[pl.ds(base, BPW)], idx_o)

    lane = lax.iota(jnp.int32, L)

    def issue(c):
        p = c % 2
        isl = pl.ds(c * CHUNK, CHUNK)
        return [
            pltpu.async_copy(ere_h.at[idx_s.at[isl]], sre_b.at[p], sem.at[p]),
            pltpu.async_copy(eim_h.at[idx_s.at[isl]], sim_b.at[p], sem.at[p]),
            pltpu.async_copy(ere_h.at[idx_o.at[isl]], ore_b.at[p], sem.at[p]),
            pltpu.async_copy(eim_h.at[idx_o.at[isl]], oim_b.at[p], sem.at[p]),
            pltpu.async_copy(rre_h.at[idx_r.at[isl]], rre_b.at[p], sem.at[p]),
            pltpu.async_copy(rim_h.at[idx_r.at[isl]], rim_b.at[p], sem.at[p]),
        ]

    pending = {0: issue(0)}
    for c in range(NCHUNK):
        p = c % 2
        if c + 1 < NCHUNK:
            pending[c + 1] = issue(c + 1)
        for dsc in pending.pop(c):
            dsc.wait()

        for g in range(CHUNK // L):
            rows = lane + (g * L)

            def dim_body(i, acc):
                d0 = i * UNROLL
                for u in range(UNROLL):
                    col = jnp.zeros((L,), jnp.int32) + (d0 + u)
                    sre = plsc.load_gather(sre_b.at[p], [rows, col])
                    sim = plsc.load_gather(sim_b.at[p], [rows, col])
                    orr = plsc.load_gather(ore_b.at[p], [rows, col])
                    oim = plsc.load_gather(oim_b.at[p], [rows, col])
                    rr = plsc.load_gather(rre_b.at[p], [rows, col])
                    ri = plsc.load_gather(rim_b.at[p], [rows, col])
                    t1 = rr * orr + ri * oim
                    t2 = rr * oim - ri * orr
                    acc = acc + sre * t1 + sim * t2
                return acc

            acc = lax.fori_loop(0, D // UNROLL, dim_body,
                                jnp.zeros((L,), jnp.float32))
            out_v[pl.ds(c * CHUNK + g * L, L)] = acc

    pltpu.sync_copy(out_v, out_h.at[pl.ds(base, BPW)])


@jax.jit
def _score(s, r, o, E_re, E_im, R_re, R_im):
    mesh = plsc.VectorSubcoreMesh(core_axis_name="c", subcore_axis_name="s")
    f = functools.partial(
        pl.kernel,
        out_type=jax.ShapeDtypeStruct((B,), jnp.float32),
        mesh=mesh,
        compiler_params=pltpu.CompilerParams(needs_layout_passes=False),
        scratch_types=[
            pltpu.VMEM((BPW,), jnp.int32),
            pltpu.VMEM((BPW,), jnp.int32),
            pltpu.VMEM((BPW,), jnp.int32),
            pltpu.VMEM((2, CHUNK, D), jnp.float32),
            pltpu.VMEM((2, CHUNK, D), jnp.float32),
            pltpu.VMEM((2, CHUNK, D), jnp.float32),
            pltpu.VMEM((2, CHUNK, D), jnp.float32),
            pltpu.VMEM((2, CHUNK, D), jnp.float32),
            pltpu.VMEM((2, CHUNK, D), jnp.float32),
            pltpu.VMEM((BPW,), jnp.float32),
            pltpu.SemaphoreType.DMA((2,)),
        ],
    )(_body)
    return f(s, r, o, E_re, E_im, R_re, R_im)


def kernel(s, r, o, t, E_re, E_im, R_re, R_im):
    del t
    return _score(s, r, o, E_re, E_im, R_re, R_im)


# speedup vs baseline: 4.7339x; 3.8299x over previous
import functools

import jax
import jax.numpy as jnp
from jax import lax
from jax.experimental import pallas as pl
from jax.experimental.pallas import tpu as pltpu
from jax.experimental.pallas import tpu_sc as plsc

NC = 2
NS = 16
L = 16
NW = NC * NS

B = 16384
D = 128
BPW = B // NW
CHUNK = 64
NCHUNK = BPW // CHUNK


UNROLL = 4


def _body(s_h, r_h, o_h, ere_h, eim_h, rre_h, rim_h, out_h,
          idx_s, idx_r, idx_o,
          sre_b, sim_b, ore_b, oim_b, rre_b, rim_b,
          out_v, sem):
    wid = lax.axis_index("s") * NC + lax.axis_index("c")
    base = wid * BPW

    pltpu.sync_copy(s_h.at[pl.ds(base, BPW)], idx_s)
    pltpu.sync_copy(r_h.at[pl.ds(base, BPW)], idx_r)
    pltpu.sync_copy(o_h.at[pl.ds(base, BPW)], idx_o)

    lane = lax.iota(jnp.int32, L)

    def issue(c):
        p = c % 2
        isl = pl.ds(c * CHUNK, CHUNK)
        return [
            pltpu.async_copy(ere_h.at[idx_s.at[isl]], sre_b.at[p], sem.at[p]),
            pltpu.async_copy(eim_h.at[idx_s.at[isl]], sim_b.at[p], sem.at[p]),
            pltpu.async_copy(ere_h.at[idx_o.at[isl]], ore_b.at[p], sem.at[p]),
            pltpu.async_copy(eim_h.at[idx_o.at[isl]], oim_b.at[p], sem.at[p]),
            pltpu.async_copy(rre_h.at[idx_r.at[isl]], rre_b.at[p], sem.at[p]),
            pltpu.async_copy(rim_h.at[idx_r.at[isl]], rim_b.at[p], sem.at[p]),
        ]

    pending = {0: issue(0)}
    for c in range(NCHUNK):
        p = c % 2
        if c + 1 < NCHUNK:
            pending[c + 1] = issue(c + 1)
        for dsc in pending.pop(c):
            dsc.wait()

        for g in range(CHUNK // L):
            def elem_body(i, scores):
                row = g * L + i

                def acc_col(j):
                    dsl = pl.ds(j * L, L)
                    sre = sre_b[p, row, dsl]
                    sim = sim_b[p, row, dsl]
                    orr = ore_b[p, row, dsl]
                    oim = oim_b[p, row, dsl]
                    rr = rre_b[p, row, dsl]
                    ri = rim_b[p, row, dsl]
                    t1 = rr * orr + ri * oim
                    t2 = rr * oim - ri * orr
                    return sre * t1 + sim * t2

                acc = acc_col(0)
                for j in range(1, D // L):
                    acc = acc + acc_col(j)
                tot = jnp.sum(acc)
                return jnp.where(lane == i, tot, scores)

            scores = lax.fori_loop(0, L, elem_body,
                                   jnp.zeros((L,), jnp.float32))
            out_v[pl.ds(c * CHUNK + g * L, L)] = scores

    pltpu.sync_copy(out_v, out_h.at[pl.ds(base, BPW)])


@jax.jit
def _score(s, r, o, E_re, E_im, R_re, R_im):
    mesh = plsc.VectorSubcoreMesh(core_axis_name="c", subcore_axis_name="s")
    f = functools.partial(
        pl.kernel,
        out_type=jax.ShapeDtypeStruct((B,), jnp.float32),
        mesh=mesh,
        compiler_params=pltpu.CompilerParams(needs_layout_passes=False),
        scratch_types=[
            pltpu.VMEM((BPW,), jnp.int32),
            pltpu.VMEM((BPW,), jnp.int32),
            pltpu.VMEM((BPW,), jnp.int32),
            pltpu.VMEM((2, CHUNK, D), jnp.float32),
            pltpu.VMEM((2, CHUNK, D), jnp.float32),
            pltpu.VMEM((2, CHUNK, D), jnp.float32),
            pltpu.VMEM((2, CHUNK, D), jnp.float32),
            pltpu.VMEM((2, CHUNK, D), jnp.float32),
            pltpu.VMEM((2, CHUNK, D), jnp.float32),
            pltpu.VMEM((BPW,), jnp.float32),
            pltpu.SemaphoreType.DMA((2,)),
        ],
    )(_body)
    return f(s, r, o, E_re, E_im, R_re, R_im)


def kernel(s, r, o, t, E_re, E_im, R_re, R_im):
    del t
    return _score(s, r, o, E_re, E_im, R_re, R_im)


# speedup vs baseline: 4.7412x; 1.0015x over previous
import functools

import jax
import jax.numpy as jnp
from jax import lax
from jax.experimental import pallas as pl
from jax.experimental.pallas import tpu as pltpu
from jax.experimental.pallas import tpu_sc as plsc

NC = 2
NS = 16
L = 16
NW = NC * NS

B = 16384
D = 128
BPW = B // NW
CHUNK = 64
NCHUNK = BPW // CHUNK


UNROLL = 4


def _body(s_h, r_h, o_h, ere_h, eim_h, rre_h, rim_h, out_h,
          idx_s, idx_r, idx_o,
          sre_b, sim_b, ore_b, oim_b, rre_b, rim_b,
          out_v, sem):
    wid = lax.axis_index("s") * NC + lax.axis_index("c")
    base = wid * BPW

    pltpu.sync_copy(s_h.at[pl.ds(base, BPW)], idx_s)
    pltpu.sync_copy(r_h.at[pl.ds(base, BPW)], idx_r)
    pltpu.sync_copy(o_h.at[pl.ds(base, BPW)], idx_o)

    lane = lax.iota(jnp.int32, L)

    def issue(c):
        p = c % 2
        isl = pl.ds(c * CHUNK, CHUNK)
        return [
            pltpu.async_copy(ere_h.at[idx_s.at[isl]], sre_b.at[p], sem.at[p]),
            pltpu.async_copy(eim_h.at[idx_s.at[isl]], sim_b.at[p], sem.at[p]),
            pltpu.async_copy(ere_h.at[idx_o.at[isl]], ore_b.at[p], sem.at[p]),
            pltpu.async_copy(eim_h.at[idx_o.at[isl]], oim_b.at[p], sem.at[p]),
            pltpu.async_copy(rre_h.at[idx_r.at[isl]], rre_b.at[p], sem.at[p]),
            pltpu.async_copy(rim_h.at[idx_r.at[isl]], rim_b.at[p], sem.at[p]),
        ]

    pending = {0: issue(0)}
    for c in range(NCHUNK):
        p = c % 2
        if c + 1 < NCHUNK:
            pending[c + 1] = issue(c + 1)
        for dsc in pending.pop(c):
            dsc.wait()

        for g in range(CHUNK // L):
            def elem_body(i, scores):
                row = g * L + i

                def acc_col(j):
                    dsl = pl.ds(j * L, L)
                    sre = sre_b[p, row, dsl]
                    sim = sim_b[p, row, dsl]
                    orr = ore_b[p, row, dsl]
                    oim = oim_b[p, row, dsl]
                    rr = rre_b[p, row, dsl]
                    ri = rim_b[p, row, dsl]
                    t1 = rr * orr + ri * oim
                    t2 = rr * oim - ri * orr
                    return sre * t1 + sim * t2

                acc = acc_col(0)
                for j in range(1, D // L):
                    acc = acc + acc_col(j)
                tot = jnp.sum(acc)
                return jnp.where(lane == i, tot, scores)

            scores = lax.fori_loop(0, L, elem_body,
                                   jnp.zeros((L,), jnp.float32))
            out_v[pl.ds(c * CHUNK + g * L, L)] = scores

    pltpu.sync_copy(out_v, out_h.at[pl.ds(base, BPW)])


@jax.jit
def _score(s, r, o, E_re, E_im, R_re, R_im):
    mesh = plsc.VectorSubcoreMesh(core_axis_name="c", subcore_axis_name="s")
    f = functools.partial(
        pl.kernel,
        out_type=jax.ShapeDtypeStruct((B,), jnp.float32),
        mesh=mesh,
        compiler_params=pltpu.CompilerParams(
            needs_layout_passes=False,
            disable_bounds_checks=True,
            disable_semaphore_checks=True,
        ),
        scratch_types=[
            pltpu.VMEM((BPW,), jnp.int32),
            pltpu.VMEM((BPW,), jnp.int32),
            pltpu.VMEM((BPW,), jnp.int32),
            pltpu.VMEM((2, CHUNK, D), jnp.float32),
            pltpu.VMEM((2, CHUNK, D), jnp.float32),
            pltpu.VMEM((2, CHUNK, D), jnp.float32),
            pltpu.VMEM((2, CHUNK, D), jnp.float32),
            pltpu.VMEM((2, CHUNK, D), jnp.float32),
            pltpu.VMEM((2, CHUNK, D), jnp.float32),
            pltpu.VMEM((BPW,), jnp.float32),
            pltpu.SemaphoreType.DMA((2,)),
        ],
    )(_body)
    return f(s, r, o, E_re, E_im, R_re, R_im)


def kernel(s, r, o, t, E_re, E_im, R_re, R_im):
    del t
    return _score(s, r, o, E_re, E_im, R_re, R_im)


# speedup vs baseline: 5.4883x; 1.1576x over previous
import functools

import jax
import jax.numpy as jnp
from jax import lax
from jax.experimental import pallas as pl
from jax.experimental.pallas import tpu as pltpu
from jax.experimental.pallas import tpu_sc as plsc

NC = 2
NS = 16
L = 16
NW = NC * NS

B = 16384
D = 128
BPW = B // NW
CHUNK = 64
NCHUNK = BPW // CHUNK


UNROLL = 4


def _body(s_h, r_h, o_h, ere_h, eim_h, rre_h, rim_h, out_h,
          idx_s, idx_r, idx_o,
          sre_b, sim_b, ore_b, oim_b, rre_b, rim_b,
          out_v, sem):
    wid = lax.axis_index("s") * NC + lax.axis_index("c")
    base = wid * BPW

    pltpu.sync_copy(s_h.at[pl.ds(base, BPW)], idx_s)
    pltpu.sync_copy(r_h.at[pl.ds(base, BPW)], idx_r)
    pltpu.sync_copy(o_h.at[pl.ds(base, BPW)], idx_o)

    lane = lax.iota(jnp.int32, L)

    def copies(c, p):
        isl = pl.ds(c * CHUNK, CHUNK)
        return [
            (ere_h.at[idx_s.at[isl]], sre_b.at[p]),
            (eim_h.at[idx_s.at[isl]], sim_b.at[p]),
            (ere_h.at[idx_o.at[isl]], ore_b.at[p]),
            (eim_h.at[idx_o.at[isl]], oim_b.at[p]),
            (rre_h.at[idx_r.at[isl]], rre_b.at[p]),
            (rim_h.at[idx_r.at[isl]], rim_b.at[p]),
        ]

    def issue(c, p):
        for src, dst in copies(c, p):
            pltpu.async_copy(src, dst, sem.at[p])

    def drain(c, p):
        for src, dst in copies(c, p):
            pltpu.make_async_copy(src, dst, sem.at[p]).wait()

    def compute(c, p):
        for g in range(CHUNK // L):
            def elem_body(i, scores):
                row = g * L + i

                def acc_col(j):
                    dsl = pl.ds(j * L, L)
                    sre = sre_b[p, row, dsl]
                    sim = sim_b[p, row, dsl]
                    orr = ore_b[p, row, dsl]
                    oim = oim_b[p, row, dsl]
                    rr = rre_b[p, row, dsl]
                    ri = rim_b[p, row, dsl]
                    t1 = rr * orr + ri * oim
                    t2 = rr * oim - ri * orr
                    return sre * t1 + sim * t2

                acc = acc_col(0)
                for j in range(1, D // L):
                    acc = acc + acc_col(j)
                tot = jnp.sum(acc)
                return jnp.where(lane == i, tot, scores)

            scores = lax.fori_loop(0, L, elem_body,
                                   jnp.zeros((L,), jnp.float32))
            out_v[pl.ds(c * CHUNK + g * L, L)] = scores

    issue(0, 0)
    issue(1, 1)

    def pair_body(k, carry):
        c0 = 2 * k
        for p in range(2):
            c = c0 + p
            drain(c, p)
            compute(c, p)

            @pl.when(k < NCHUNK // 2 - 1)
            def _():
                issue(c + 2, p)
        return carry

    lax.fori_loop(0, NCHUNK // 2, pair_body, 0)

    pltpu.sync_copy(out_v, out_h.at[pl.ds(base, BPW)])


@jax.jit
def _score(s, r, o, E_re, E_im, R_re, R_im):
    mesh = plsc.VectorSubcoreMesh(core_axis_name="c", subcore_axis_name="s")
    f = functools.partial(
        pl.kernel,
        out_type=jax.ShapeDtypeStruct((B,), jnp.float32),
        mesh=mesh,
        compiler_params=pltpu.CompilerParams(
            needs_layout_passes=False,
            disable_bounds_checks=True,
            disable_semaphore_checks=True,
        ),
        scratch_types=[
            pltpu.VMEM((BPW,), jnp.int32),
            pltpu.VMEM((BPW,), jnp.int32),
            pltpu.VMEM((BPW,), jnp.int32),
            pltpu.VMEM((2, CHUNK, D), jnp.float32),
            pltpu.VMEM((2, CHUNK, D), jnp.float32),
            pltpu.VMEM((2, CHUNK, D), jnp.float32),
            pltpu.VMEM((2, CHUNK, D), jnp.float32),
            pltpu.VMEM((2, CHUNK, D), jnp.float32),
            pltpu.VMEM((2, CHUNK, D), jnp.float32),
            pltpu.VMEM((BPW,), jnp.float32),
            pltpu.SemaphoreType.DMA((2,)),
        ],
    )(_body)
    return f(s, r, o, E_re, E_im, R_re, R_im)


def kernel(s, r, o, t, E_re, E_im, R_re, R_im):
    del t
    return _score(s, r, o, E_re, E_im, R_re, R_im)
